# Initial kernel scaffold; baseline (speedup 1.0000x reference)
import jax
import jax.numpy as jnp
from jax.experimental import pallas as pl


def kernel(x, rows, cols, a_vals, rel_ids, W1, b1, bn1_gamma, bn1_beta, bn2_gamma, bn2_beta, rel_coeffs, coeff_kernel, W2, b2):
    raise NotImplementedError("write your pallas kernel here")


# speedup vs baseline: 4.0307x; 4.0307x over previous
import functools
import math

import jax
import jax.numpy as jnp
from jax import lax
from jax.experimental import pallas as pl
from jax.experimental.pallas import tpu as pltpu
from jax.experimental.pallas import tpu_sc as plsc

N = 10000
E = 320000
D = 128
R = 200

NC = 2
NS = 16
F = D // NC
NCH = N // 8
CH_BASE = NCH // NS
CH_EXTRA = NCH - CH_BASE * NS
ROWS_BIG = (CH_BASE + 1) * 8
ROWS_SMALL = CH_BASE * 8
EW = 128
NWIN = E // EW
WIN_BASE = NWIN // NS
WIN_EXTRA = NWIN - WIN_BASE * NS

RPAD = 256

C_BN = float(1.0 / math.sqrt(1.0 + 1e-3))

_mesh = functools.partial(
    plsc.VectorSubcoreMesh, core_axis_name="c", subcore_axis_name="s",
    num_cores=NC, num_subcores=NS)


def _zero_gath(gath):
    def zrow(r, _):
        for k in range(F // 16):
            gath[r, pl.ds(k * 16, 16)] = jnp.zeros((16,), jnp.float32)
        return 0
    lax.fori_loop(0, EW, zrow, 0)


def _stage_slice(h_hbm, c, r0, nrows, h_sp, acc_sp, gath):
    pltpu.sync_copy(h_hbm.at[c, pl.ds(r0, nrows), :],
                    h_sp.at[pl.ds(r0, nrows), :])
    nfull, tail = nrows // EW, nrows % EW
    for j in range(nfull):
        pltpu.sync_copy(gath, acc_sp.at[pl.ds(r0 + j * EW, EW), :])
    if tail:
        pltpu.sync_copy(gath.at[pl.ds(0, tail), :],
                        acc_sp.at[pl.ds(r0 + nfull * EW, tail), :])


def _scale_window(gath, valb):
    def edge(e, _):
        esplat = jnp.full((16,), e, jnp.int32)
        vsp = plsc.load_gather(valb, [esplat])
        for k in range(F // 16):
            sl = pl.ds(k * 16, 16)
            gath[e, sl] = gath[e, sl] * vsp
        return 0
    lax.fori_loop(0, EW, edge, 0)


def _make_spmm(use_rel):
    def body(*refs):
        if use_rel:
            (h_hbm, cols_h, rows_h, vals_h, rel_h, rc_h, out_hbm,
             h_sp, acc_sp, colb, rowb, valb, gath, relb, rctab) = refs
        else:
            (h_hbm, cols_h, rows_h, vals_h, out_hbm,
             h_sp, acc_sp, colb, rowb, valb, gath) = refs
            relb = rctab = rel_h = rc_h = None

        c = lax.axis_index("c")
        s = lax.axis_index("s")
        r0 = (s * CH_BASE + jnp.minimum(s, CH_EXTRA)) * 8
        big = s < CH_EXTRA

        _zero_gath(gath)
        pl.when(big)(lambda: _stage_slice(h_hbm, c, r0, ROWS_BIG,
                                          h_sp, acc_sp, gath))
        pl.when(jnp.logical_not(big))(lambda: _stage_slice(
            h_hbm, c, r0, ROWS_SMALL, h_sp, acc_sp, gath))
        if use_rel:
            pltpu.sync_copy(rc_h, rctab)
        plsc.subcore_barrier()

        nwin = WIN_BASE + jnp.where(s < WIN_EXTRA, 1, 0)
        win0 = s * WIN_BASE + jnp.minimum(s, WIN_EXTRA)

        def win(w, _):
            base = (win0 + w) * EW
            pltpu.sync_copy(cols_h.at[pl.ds(base, EW)], colb)
            pltpu.sync_copy(rows_h.at[pl.ds(base, EW)], rowb)
            pltpu.sync_copy(vals_h.at[pl.ds(base, EW)], valb)
            if use_rel:
                pltpu.sync_copy(rel_h.at[pl.ds(base, EW)], relb)
                for j in range(EW // 16):
                    sl = pl.ds(j * 16, 16)
                    rc = plsc.load_gather(rctab, [relb[sl]])
                    valb[sl] = valb[sl] / (rc + 1.0)
            pltpu.sync_copy(h_sp.at[colb], gath)
            _scale_window(gath, valb)
            pltpu.sync_copy(gath, acc_sp.at[rowb], add=True)
            return 0

        lax.fori_loop(0, nwin, win, 0)

        plsc.subcore_barrier()
        pl.when(big)(lambda: pltpu.sync_copy(
            acc_sp.at[pl.ds(r0, ROWS_BIG), :],
            out_hbm.at[c, pl.ds(r0, ROWS_BIG), :]))
        pl.when(jnp.logical_not(big))(lambda: pltpu.sync_copy(
            acc_sp.at[pl.ds(r0, ROWS_SMALL), :],
            out_hbm.at[c, pl.ds(r0, ROWS_SMALL), :]))

    scratch = [
        pltpu.VMEM_SHARED((N, F), jnp.float32),
        pltpu.VMEM_SHARED((N, F), jnp.float32),
        pltpu.VMEM((EW,), jnp.int32),
        pltpu.VMEM((EW,), jnp.int32),
        pltpu.VMEM((EW,), jnp.float32),
        pltpu.VMEM((EW, F), jnp.float32),
    ]
    if use_rel:
        scratch += [
            pltpu.VMEM((EW,), jnp.int32),
            pltpu.VMEM((RPAD,), jnp.float32),
        ]

    return pl.kernel(
        body,
        out_type=jax.ShapeDtypeStruct((NC, N, F), jnp.float32),
        mesh=_mesh(),
        scratch_types=scratch,
        compiler_params=pltpu.CompilerParams(needs_layout_passes=False),
    )


_spmm_plain = _make_spmm(use_rel=False)
_spmm_rel = _make_spmm(use_rel=True)


BR = 1000


def _k1_body(x_ref, w1_ref, g_ref, b_ref, out_ref):
    xb = x_ref[...]
    h = (g_ref[...] * C_BN) * xb + b_ref[...]
    t = jnp.dot(h, w1_ref[...], preferred_element_type=jnp.float32)
    out_ref[0] = t[:, :F]
    out_ref[1] = t[:, F:]


def _k2_body(s1_ref, b1_ref, g2_ref, be2_ref, ck_ref, y1_ref, h2_ref, p_ref):
    s1 = jnp.concatenate([s1_ref[0], s1_ref[1]], axis=1)
    y1 = jnp.tanh(s1 + b1_ref[...])
    h2 = (g2_ref[...] * C_BN) * y1 + be2_ref[...]
    y1_ref[...] = y1
    h2_ref[0] = h2[:, :F]
    h2_ref[1] = h2[:, F:]
    p_ref[...] = h2 * (ck_ref[...] + 1.0)


def _l2_parts(v):
    sq = jnp.sum(v * v, axis=1, keepdims=True)
    den = jnp.maximum(sq, 1e-12)
    u = v / jnp.sqrt(den)
    return u, sq / den


def _k3_body(s2_ref, p_ref, w2_ref, b2_ref, x_ref, y1_ref, out_ref):
    m = jnp.concatenate([s2_ref[0], s2_ref[1]], axis=1) + p_ref[...]
    y2 = jnp.dot(m, w2_ref[...], preferred_element_type=jnp.float32) + b2_ref[...]
    u1, n1 = _l2_parts(x_ref[...])
    u2, n2 = _l2_parts(y1_ref[...])
    u3, n3 = _l2_parts(y2)
    inv = 1.0 / jnp.sqrt(jnp.maximum(n1 + n2 + n3, 1e-12))
    out_ref[:, 0:D] = u1 * inv
    out_ref[:, D:2 * D] = u2 * inv
    out_ref[:, 2 * D:3 * D] = u3 * inv


def _row_spec(width):
    return pl.BlockSpec((BR, width), lambda i: (i, 0))


def _full_spec(shape):
    return pl.BlockSpec(shape, lambda i: tuple(0 for _ in shape))


_split_spec = pl.BlockSpec((NC, BR, F), lambda i: (0, i, 0))


def kernel(x, rows, cols, a_vals, rel_ids, W1, b1, bn1_gamma, bn1_beta,
           bn2_gamma, bn2_beta, rel_coeffs, coeff_kernel, W2, b2):
    f32 = jnp.float32
    b1r = b1.reshape(1, D)
    b2r = b2.reshape(1, D)
    g1 = bn1_gamma.reshape(1, D)
    be1 = bn1_beta.reshape(1, D)
    g2 = bn2_gamma.reshape(1, D)
    be2 = bn2_beta.reshape(1, D)
    rc_pad = jnp.zeros((RPAD,), f32).at[:R].set(rel_coeffs)

    grid = (N // BR,)

    t2 = pl.pallas_call(
        _k1_body,
        grid=grid,
        in_specs=[_row_spec(D), _full_spec((D, D)), _full_spec((1, D)),
                  _full_spec((1, D))],
        out_specs=_split_spec,
        out_shape=jax.ShapeDtypeStruct((NC, N, F), f32),
    )(x, W1, g1, be1)

    s1 = _spmm_plain(t2, cols, rows, a_vals)

    y1, h2, p = pl.pallas_call(
        _k2_body,
        grid=grid,
        in_specs=[_split_spec, _full_spec((1, D)), _full_spec((1, D)),
                  _full_spec((1, D)), _row_spec(1)],
        out_specs=[_row_spec(D), _split_spec, _row_spec(D)],
        out_shape=[jax.ShapeDtypeStruct((N, D), f32),
                   jax.ShapeDtypeStruct((NC, N, F), f32),
                   jax.ShapeDtypeStruct((N, D), f32)],
    )(s1, b1r, g2, be2, coeff_kernel)

    s2 = _spmm_rel(h2, cols, rows, a_vals, rel_ids, rc_pad)

    out = pl.pallas_call(
        _k3_body,
        grid=grid,
        in_specs=[_split_spec, _row_spec(D), _full_spec((D, D)),
                  _full_spec((1, D)), _row_spec(D), _row_spec(D)],
        out_specs=_row_spec(3 * D),
        out_shape=jax.ShapeDtypeStruct((N, 3 * D), f32),
    )(s2, p, W2, b2r, x, y1)

    return out
